# baseline (speedup 1.0000x reference)
import functools

import jax
import jax.numpy as jnp
from jax import lax
from jax.experimental import pallas as pl
from jax.experimental.pallas import tpu as pltpu
from jax.experimental.pallas import tpu_sc as plsc

N = 100000
D = 128
NUM_SEG = 1024
BLK = 1024
NB = 98
NPAD = NB * BLK
NW = 32
CHUNK = NPAD // NW
GRP = CHUNK // 16
SCHUNK = 112
NSC = CHUNK // SCHUNK

NEG_INF = float("-inf")


def _gate_max_kernel(x_ref, wcol_ref, b_ref, bcol_ref, gate_ref, segmax_ref):
  i = pl.program_id(0)
  xb = x_ref[...]
  gate = jnp.dot(xb, wcol_ref[...], preferred_element_type=jnp.float32)
  gate = gate + b_ref[0, 0]
  gate_ref[...] = gate

  seg = jax.lax.broadcasted_iota(jnp.int32, (BLK, NUM_SEG), 1)
  mask = bcol_ref[...] == seg
  vals = jnp.where(mask, gate, NEG_INF)
  colmax = jnp.max(vals, axis=0, keepdims=True)

  @pl.when(i == 0)
  def _init():
    segmax_ref[...] = jnp.full((1, NUM_SEG), NEG_INF, jnp.float32)

  segmax_ref[...] = jnp.maximum(segmax_ref[...], colmax)


def _sc_exp_segsum(gate_hbm, idx_hbm, segmax_hbm, zeros_hbm, e_hbm, parts_hbm,
                   gate_v, idx_v, e_v, rows_v, segmax_v, table_sh):
  cid = lax.axis_index("c")
  sid = lax.axis_index("s")
  wid = sid * 2 + cid
  base = wid * CHUNK

  pltpu.sync_copy(gate_hbm.at[pl.ds(base, CHUNK)], gate_v)
  pltpu.sync_copy(idx_hbm.at[pl.ds(base, CHUNK)], idx_v)
  pltpu.sync_copy(segmax_hbm, segmax_v)

  @pl.when(sid == 0)
  def _init_table():
    pltpu.sync_copy(zeros_hbm, table_sh)

  plsc.subcore_barrier()

  lane0 = jnp.zeros((16,), jnp.int32)
  ri0 = jax.lax.broadcasted_iota(jnp.int32, (16,), 0)

  def body(g, carry):
    off = g * 16
    idxv = idx_v[pl.ds(off, 16)]
    gv = gate_v[pl.ds(off, 16)]
    mx = plsc.load_gather(segmax_v, [idxv])
    e = jnp.exp(gv - mx)
    e_v[pl.ds(off, 16)] = e
    plsc.store_scatter(rows_v, [ri0 + off, lane0], e)
    return carry

  lax.fori_loop(0, GRP, body, 0)

  def sbody(c, carry):
    soff = c * SCHUNK
    pltpu.sync_copy(rows_v.at[pl.ds(soff, SCHUNK)],
                    table_sh.at[idx_v.at[pl.ds(soff, SCHUNK)]], add=True)
    return carry

  lax.fori_loop(0, NSC, sbody, 0)

  pltpu.sync_copy(e_v, e_hbm.at[pl.ds(base, CHUNK)])

  plsc.subcore_barrier()

  @pl.when(sid == 0)
  def _flush():
    pltpu.sync_copy(table_sh, parts_hbm.at[cid])


def _attn_out_kernel(x_ref, g_ref, bcol_ref, brow_ref, segsum_ref, attn_ref,
                     out_ref):
  i = pl.program_id(0)
  seg = jax.lax.broadcasted_iota(jnp.int32, (BLK, NUM_SEG), 1)
  mask = bcol_ref[...] == seg
  ss = jnp.sum(jnp.where(mask, segsum_ref[...], 0.0), axis=1, keepdims=True)
  a = g_ref[...] / (ss + 1e-16)
  attn_ref[...] = a
  w = a * jnp.maximum(x_ref[...], 0.0)

  segT = jax.lax.broadcasted_iota(jnp.int32, (NUM_SEG, BLK), 0)
  maskT = (brow_ref[0] == segT).astype(jnp.bfloat16)
  contrib = jnp.dot(maskT, w.astype(jnp.bfloat16),
                    preferred_element_type=jnp.float32)

  @pl.when(i == 0)
  def _init():
    out_ref[...] = jnp.zeros((NUM_SEG, D), jnp.float32)

  out_ref[...] = out_ref[...] + contrib


@jax.jit
def kernel(x, W, b, batch):
  pad = NPAD - N
  xp = jnp.pad(x, ((0, pad), (0, 0)))
  bi = batch.astype(jnp.int32)
  bip = jnp.pad(bi, (0, pad), constant_values=NUM_SEG)
  bcol = bip.reshape(NPAD, 1)
  brow = bip.reshape(NB, 1, BLK)
  wcol = W.reshape(1, D).T
  b2 = b.reshape(1, 1)

  gate, segmax = pl.pallas_call(
      _gate_max_kernel,
      grid=(NB,),
      in_specs=[
          pl.BlockSpec((BLK, D), lambda i: (i, 0)),
          pl.BlockSpec((D, 1), lambda i: (0, 0)),
          pl.BlockSpec((1, 1), lambda i: (0, 0)),
          pl.BlockSpec((BLK, 1), lambda i: (i, 0)),
      ],
      out_specs=[
          pl.BlockSpec((BLK, 1), lambda i: (i, 0)),
          pl.BlockSpec((1, NUM_SEG), lambda i: (0, 0)),
      ],
      out_shape=[
          jax.ShapeDtypeStruct((NPAD, 1), jnp.float32),
          jax.ShapeDtypeStruct((1, NUM_SEG), jnp.float32),
      ],
  )(xp, wcol, b2, bcol)

  segmax_t = jnp.concatenate(
      [segmax[0], jnp.zeros((1,), jnp.float32)])
  zeros_t = jnp.zeros((NUM_SEG + 1, 16), jnp.float32)

  sc_fn = functools.partial(
      pl.kernel,
      mesh=plsc.VectorSubcoreMesh(core_axis_name="c", subcore_axis_name="s"),
      compiler_params=pltpu.CompilerParams(
          needs_layout_passes=False, use_tc_tiling_on_sc=False),
      out_type=[
          jax.ShapeDtypeStruct((NPAD,), jnp.float32),
          jax.ShapeDtypeStruct((2, NUM_SEG + 1, 16), jnp.float32),
      ],
      scratch_types=[
          pltpu.VMEM((CHUNK,), jnp.float32),
          pltpu.VMEM((CHUNK,), jnp.int32),
          pltpu.VMEM((CHUNK,), jnp.float32),
          pltpu.VMEM((CHUNK, 16), jnp.float32),
          pltpu.VMEM((NUM_SEG + 1,), jnp.float32),
          pltpu.VMEM_SHARED((NUM_SEG + 1, 16), jnp.float32),
      ],
  )(_sc_exp_segsum)
  g1d, parts = sc_fn(gate.reshape(NPAD), bip, segmax_t, zeros_t)

  segsum = (parts[0, :NUM_SEG, 0] + parts[1, :NUM_SEG, 0]).reshape(1, NUM_SEG)
  g = g1d.reshape(NPAD, 1)

  attn, out = pl.pallas_call(
      _attn_out_kernel,
      grid=(NB,),
      in_specs=[
          pl.BlockSpec((BLK, D), lambda i: (i, 0)),
          pl.BlockSpec((BLK, 1), lambda i: (i, 0)),
          pl.BlockSpec((BLK, 1), lambda i: (i, 0)),
          pl.BlockSpec((1, 1, BLK), lambda i: (i, 0, 0)),
          pl.BlockSpec((1, NUM_SEG), lambda i: (0, 0)),
      ],
      out_specs=[
          pl.BlockSpec((BLK, 1), lambda i: (i, 0)),
          pl.BlockSpec((NUM_SEG, D), lambda i: (0, 0)),
      ],
      out_shape=[
          jax.ShapeDtypeStruct((NPAD, 1), jnp.float32),
          jax.ShapeDtypeStruct((NUM_SEG, D), jnp.float32),
      ],
  )(xp, g, bcol, brow, segsum)

  return out, attn[:N]


# speedup vs baseline: 1.3601x; 1.3601x over previous
import functools

import jax
import jax.numpy as jnp
from jax import lax
from jax.experimental import pallas as pl
from jax.experimental.pallas import tpu as pltpu
from jax.experimental.pallas import tpu_sc as plsc

N = 100000
D = 128
NUM_SEG = 1024
BLK = 1024
NB = 98
NPAD = NB * BLK
NW = 32
CHUNK = NPAD // NW
GRP = CHUNK // 16
SCHUNK = 112
NSC = CHUNK // SCHUNK

NEG_INF = float("-inf")


def _gate_max_kernel(x_ref, wcol_ref, b_ref, brow_ref, gate3_ref, segmax_ref):
  i = pl.program_id(0)
  xb = x_ref[...]
  gate = jnp.dot(xb, wcol_ref[...], preferred_element_type=jnp.float32)
  gate = gate + b_ref[0, 0]
  gate_row = gate.reshape(1, BLK)
  gate3_ref[0] = gate_row

  seg = jax.lax.broadcasted_iota(jnp.int32, (NUM_SEG, BLK), 0)
  mask = brow_ref[0] == seg
  vals = jnp.where(mask, gate_row, NEG_INF)
  rowmax = jnp.max(vals, axis=1, keepdims=True)

  @pl.when(i == 0)
  def _init():
    segmax_ref[...] = jnp.full((NUM_SEG, 1), NEG_INF, jnp.float32)

  segmax_ref[...] = jnp.maximum(segmax_ref[...], rowmax)


def _sc_exp_segsum(gate_hbm, idx_hbm, segmax_hbm, zeros_hbm, e_hbm, parts_hbm,
                   gate_v, idx_v, e_v, rows_v, segmax_v, table_sh):
  cid = lax.axis_index("c")
  sid = lax.axis_index("s")
  wid = sid * 2 + cid
  base = wid * CHUNK

  pltpu.sync_copy(gate_hbm.at[pl.ds(base, CHUNK)], gate_v)
  pltpu.sync_copy(idx_hbm.at[pl.ds(base, CHUNK)], idx_v)
  pltpu.sync_copy(segmax_hbm, segmax_v)

  @pl.when(sid == 0)
  def _init_table():
    pltpu.sync_copy(zeros_hbm, table_sh)

  plsc.subcore_barrier()

  lane0 = jnp.zeros((16,), jnp.int32)
  ri0 = jax.lax.broadcasted_iota(jnp.int32, (16,), 0)

  def body(g, carry):
    off = g * 16
    idxv = idx_v[pl.ds(off, 16)]
    gv = gate_v[pl.ds(off, 16)]
    mx = plsc.load_gather(segmax_v, [idxv])
    e = jnp.exp(gv - mx)
    e_v[pl.ds(off, 16)] = e
    plsc.store_scatter(rows_v, [ri0 + off, lane0], e)
    return carry

  lax.fori_loop(0, GRP, body, 0)

  def sbody(c, carry):
    soff = c * SCHUNK
    pltpu.sync_copy(rows_v.at[pl.ds(soff, SCHUNK)],
                    table_sh.at[idx_v.at[pl.ds(soff, SCHUNK)]], add=True)
    return carry

  lax.fori_loop(0, NSC, sbody, 0)

  pltpu.sync_copy(e_v, e_hbm.at[pl.ds(base, CHUNK)])

  plsc.subcore_barrier()

  @pl.when(sid == 0)
  def _flush():
    pltpu.sync_copy(table_sh, parts_hbm.at[cid])


def _attn_out_kernel(x_ref, e3_ref, brow_ref, segsum_ref, attn_ref, out_ref):
  i = pl.program_id(0)
  seg = jax.lax.broadcasted_iota(jnp.int32, (NUM_SEG, BLK), 0)
  mask = brow_ref[0] == seg
  ssrow = jnp.sum(jnp.where(mask, segsum_ref[...], 0.0), axis=0,
                  keepdims=True)
  a_row = e3_ref[0] / (ssrow + 1e-16)
  a_col = a_row.reshape(BLK, 1)
  attn_ref[...] = a_col

  valid = (jax.lax.broadcasted_iota(jnp.int32, (BLK, 1), 0) + i * BLK) < N
  w = jnp.where(valid, a_col * jnp.maximum(x_ref[...], 0.0), 0.0)

  contrib = jnp.dot(mask.astype(jnp.bfloat16), w.astype(jnp.bfloat16),
                    preferred_element_type=jnp.float32)

  @pl.when(i == 0)
  def _init():
    out_ref[...] = jnp.zeros((NUM_SEG, D), jnp.float32)

  out_ref[...] = out_ref[...] + contrib


@jax.jit
def kernel(x, W, b, batch):
  bi = batch.astype(jnp.int32)
  bip = jnp.pad(bi, (0, NPAD - N), constant_values=NUM_SEG)
  brow = bip.reshape(NB, 1, BLK)
  wcol = W.reshape(1, D).T
  b2 = b.reshape(1, 1)

  gate3, segmax = pl.pallas_call(
      _gate_max_kernel,
      grid=(NB,),
      in_specs=[
          pl.BlockSpec((BLK, D), lambda i: (i, 0)),
          pl.BlockSpec((D, 1), lambda i: (0, 0)),
          pl.BlockSpec((1, 1), lambda i: (0, 0)),
          pl.BlockSpec((1, 1, BLK), lambda i: (i, 0, 0)),
      ],
      out_specs=[
          pl.BlockSpec((1, 1, BLK), lambda i: (i, 0, 0)),
          pl.BlockSpec((NUM_SEG, 1), lambda i: (0, 0)),
      ],
      out_shape=[
          jax.ShapeDtypeStruct((NB, 1, BLK), jnp.float32),
          jax.ShapeDtypeStruct((NUM_SEG, 1), jnp.float32),
      ],
  )(x, wcol, b2, brow)

  segmax_t = jnp.concatenate(
      [segmax[:, 0], jnp.zeros((1,), jnp.float32)])
  zeros_t = jnp.zeros((NUM_SEG + 1, 16), jnp.float32)

  sc_fn = functools.partial(
      pl.kernel,
      mesh=plsc.VectorSubcoreMesh(core_axis_name="c", subcore_axis_name="s"),
      compiler_params=pltpu.CompilerParams(
          needs_layout_passes=False, use_tc_tiling_on_sc=False),
      out_type=[
          jax.ShapeDtypeStruct((NPAD,), jnp.float32),
          jax.ShapeDtypeStruct((2, NUM_SEG + 1, 16), jnp.float32),
      ],
      scratch_types=[
          pltpu.VMEM((CHUNK,), jnp.float32),
          pltpu.VMEM((CHUNK,), jnp.int32),
          pltpu.VMEM((CHUNK,), jnp.float32),
          pltpu.VMEM((CHUNK, 16), jnp.float32),
          pltpu.VMEM((NUM_SEG + 1,), jnp.float32),
          pltpu.VMEM_SHARED((NUM_SEG + 1, 16), jnp.float32),
      ],
  )(_sc_exp_segsum)
  g1d, parts = sc_fn(gate3.reshape(NPAD), bip, segmax_t, zeros_t)

  segsum = (parts[0, :NUM_SEG, 0] + parts[1, :NUM_SEG, 0]).reshape(NUM_SEG, 1)
  e3 = g1d.reshape(NB, 1, BLK)

  attn, out = pl.pallas_call(
      _attn_out_kernel,
      grid=(NB,),
      in_specs=[
          pl.BlockSpec((BLK, D), lambda i: (i, 0)),
          pl.BlockSpec((1, 1, BLK), lambda i: (i, 0, 0)),
          pl.BlockSpec((1, 1, BLK), lambda i: (i, 0, 0)),
          pl.BlockSpec((NUM_SEG, 1), lambda i: (0, 0)),
      ],
      out_specs=[
          pl.BlockSpec((BLK, 1), lambda i: (i, 0)),
          pl.BlockSpec((NUM_SEG, D), lambda i: (0, 0)),
      ],
      out_shape=[
          jax.ShapeDtypeStruct((N, 1), jnp.float32),
          jax.ShapeDtypeStruct((NUM_SEG, D), jnp.float32),
      ],
  )(x, e3, brow, segsum)

  return out, attn
